# initial kernel scaffold (unmeasured)
import jax
import jax.numpy as jnp
from jax import lax
from jax.experimental import pallas as pl
from jax.experimental.pallas import tpu as pltpu

N_DEV = 32


def kernel(x, w_mat, scale_x, scale_w):
    m, k_shard = x.shape
    k, n = w_mat.shape
    m_per = m // N_DEV

    def body(x_ref, w_ref, sx_ref, sw_ref, out_ref,
             xrows_ref, send_sems, recv_sems):
        me = lax.axis_index("i")

        for j in range(N_DEV):
            @pl.when(j != me)
            def _(j=j):
                rdma = pltpu.make_async_remote_copy(
                    src_ref=x_ref.at[pl.ds(j * m_per, m_per), :],
                    dst_ref=xrows_ref.at[me],
                    send_sem=send_sems.at[j],
                    recv_sem=recv_sems.at[me],
                    device_id=(j,),
                    device_id_type=pl.DeviceIdType.MESH,
                )
                rdma.start()

        @pl.when(me == me)
        def _():
            pass
        xrows_ref[me] = x_ref[pl.ds(me * m_per, m_per), :]

        for j in range(N_DEV):
            @pl.when(j != me)
            def _(j=j):
                recv = pltpu.make_async_remote_copy(
                    src_ref=x_ref.at[pl.ds(0, m_per), :],
                    dst_ref=xrows_ref.at[j],
                    send_sem=send_sems.at[j],
                    recv_sem=recv_sems.at[j],
                    device_id=(j,),
                    device_id_type=pl.DeviceIdType.MESH,
                )
                recv.wait_recv()

        acc = jnp.zeros((m_per, n), dtype=jnp.float32)
        for j in range(N_DEV):
            acc = acc + jnp.dot(
                xrows_ref[j],
                w_ref[pl.ds(j * k_shard, k_shard), :],
                preferred_element_type=jnp.float32,
            )

        scale = sx_ref[0] * sw_ref[0]
        out_ref[:, :] = jnp.maximum(acc * scale, 0.0)

        for j in range(N_DEV):
            @pl.when(j != me)
            def _(j=j):
                send = pltpu.make_async_remote_copy(
                    src_ref=x_ref.at[pl.ds(j * m_per, m_per), :],
                    dst_ref=xrows_ref.at[me],
                    send_sem=send_sems.at[j],
                    recv_sem=recv_sems.at[me],
                    device_id=(j,),
                    device_id_type=pl.DeviceIdType.MESH,
                )
                send.wait_send()

    return pl.pallas_call(
        body,
        out_shape=jax.ShapeDtypeStruct((m_per, n), jnp.float32),
        in_specs=[
            pl.BlockSpec(memory_space=pltpu.VMEM),
            pl.BlockSpec(memory_space=pltpu.VMEM),
            pl.BlockSpec(memory_space=pltpu.SMEM),
            pl.BlockSpec(memory_space=pltpu.SMEM),
        ],
        out_specs=pl.BlockSpec(memory_space=pltpu.VMEM),
        scratch_shapes=[
            pltpu.VMEM((N_DEV, m_per, k_shard), x.dtype),
            pltpu.SemaphoreType.DMA((N_DEV,)),
            pltpu.SemaphoreType.DMA((N_DEV,)),
        ],
        compiler_params=pltpu.CompilerParams(collective_id=0),
    )(x, w_mat, scale_x, scale_w)


# baseline (device time: 58056 ns/iter reference)
import jax
import jax.numpy as jnp
from jax import lax
from jax.experimental import pallas as pl
from jax.experimental.pallas import tpu as pltpu

N_DEV = 32
W_SLOTS = 4


def kernel(x, w_mat, scale_x, scale_w):
    m, k_shard = x.shape
    k, n = w_mat.shape
    m_per = m // N_DEV

    def body(x_ref, w_ref, sx_ref, sw_ref, out_ref,
             xq_ref, xrows_ref, wbuf_ref, send_sems, recv_sems, wdma_sems):
        me = lax.axis_index("i")

        xq_ref[:, :] = x_ref[:, :].astype(jnp.float8_e4m3fn)

        for j in range(N_DEV):
            @pl.when(j != me)
            def _(j=j):
                rdma = pltpu.make_async_remote_copy(
                    src_ref=xq_ref.at[pl.ds(j * m_per, m_per), :],
                    dst_ref=xrows_ref.at[me],
                    send_sem=send_sems.at[j],
                    recv_sem=recv_sems.at[me],
                    device_id=(j,),
                    device_id_type=pl.DeviceIdType.MESH,
                )
                rdma.start()

        xrows_ref[me] = xq_ref[pl.ds(me * m_per, m_per), :]

        def w_dma(j, slot):
            return pltpu.make_async_copy(
                w_ref.at[pl.ds(j * m_per, m_per), :],
                wbuf_ref.at[slot],
                wdma_sems.at[slot],
            )

        for j in range(W_SLOTS):
            w_dma(j, j).start()

        out_ref[:, :] = jnp.zeros((m_per, n), dtype=jnp.float32)

        for j in range(N_DEV):
            @pl.when(j != me)
            def _(j=j):
                recv = pltpu.make_async_remote_copy(
                    src_ref=xq_ref.at[pl.ds(0, m_per), :],
                    dst_ref=xrows_ref.at[j],
                    send_sem=send_sems.at[j],
                    recv_sem=recv_sems.at[j],
                    device_id=(j,),
                    device_id_type=pl.DeviceIdType.MESH,
                )
                recv.wait_recv()

            w_dma(j, j % W_SLOTS).wait()

            wq = wbuf_ref[j % W_SLOTS].astype(jnp.float8_e5m2)
            out_ref[:, :] += jnp.dot(
                xrows_ref[j], wq, preferred_element_type=jnp.float32,
            )

            if j + W_SLOTS < N_DEV:
                w_dma(j + W_SLOTS, j % W_SLOTS).start()

        scale = sx_ref[0] * sw_ref[0]
        out_ref[:, :] = jnp.maximum(out_ref[:, :] * scale, 0.0)

        for j in range(N_DEV):
            @pl.when(j != me)
            def _(j=j):
                send = pltpu.make_async_remote_copy(
                    src_ref=xq_ref.at[pl.ds(j * m_per, m_per), :],
                    dst_ref=xrows_ref.at[me],
                    send_sem=send_sems.at[j],
                    recv_sem=recv_sems.at[me],
                    device_id=(j,),
                    device_id_type=pl.DeviceIdType.MESH,
                )
                send.wait_send()

    return pl.pallas_call(
        body,
        out_shape=jax.ShapeDtypeStruct((m_per, n), jnp.float32),
        in_specs=[
            pl.BlockSpec(memory_space=pltpu.VMEM),
            pl.BlockSpec(memory_space=pl.ANY),
            pl.BlockSpec(memory_space=pltpu.SMEM),
            pl.BlockSpec(memory_space=pltpu.SMEM),
        ],
        out_specs=pl.BlockSpec(memory_space=pltpu.VMEM),
        scratch_shapes=[
            pltpu.VMEM((m, k_shard), jnp.float8_e4m3fn),
            pltpu.VMEM((N_DEV, m_per, k_shard), jnp.float8_e4m3fn),
            pltpu.VMEM((W_SLOTS, m_per, n), jnp.float32),
            pltpu.SemaphoreType.DMA((N_DEV,)),
            pltpu.SemaphoreType.DMA((N_DEV,)),
            pltpu.SemaphoreType.DMA((W_SLOTS,)),
        ],
    )(x, w_mat, scale_x, scale_w)


# device time: 54626 ns/iter; 1.0628x vs baseline; 1.0628x over previous
import os

import jax
import jax.numpy as jnp
from jax import lax
from jax.experimental import pallas as pl
from jax.experimental.pallas import tpu as pltpu

N_DEV = 32
W_SLOTS = 4

_VARIANT = os.environ.get("KERNEL_VARIANT", "full")


def kernel(x, w_mat, scale_x, scale_w):
    m, k_shard = x.shape
    k, n = w_mat.shape
    m_per = m // N_DEV

    def body(x_ref, w_ref, sx_ref, sw_ref, out_ref,
             xq_ref, xrows_ref, wbuf_ref, send_sems, recv_sems, wdma_sems):
        me = lax.axis_index("i")

        xq_ref[:, :] = x_ref[:, :].astype(jnp.float8_e4m3fn)

        for j in range(N_DEV):
            @pl.when(j != me)
            def _(j=j):
                rdma = pltpu.make_async_remote_copy(
                    src_ref=xq_ref.at[pl.ds(j * m_per, m_per), :],
                    dst_ref=xrows_ref.at[me],
                    send_sem=send_sems.at[j],
                    recv_sem=recv_sems.at[me],
                    device_id=(j,),
                    device_id_type=pl.DeviceIdType.MESH,
                )
                rdma.start()

        xrows_ref[me] = xq_ref[pl.ds(me * m_per, m_per), :]

        def w_dma(j, slot):
            return pltpu.make_async_copy(
                w_ref.at[pl.ds(j * m_per, m_per), :],
                wbuf_ref.at[slot],
                wdma_sems.at[slot],
            )

        for j in range(W_SLOTS):
            w_dma(j, j).start()

        out_ref[:, :] = jnp.zeros((m_per, n), dtype=jnp.float32)

        for j in range(N_DEV):
            @pl.when(j != me)
            def _(j=j):
                recv = pltpu.make_async_remote_copy(
                    src_ref=xq_ref.at[pl.ds(0, m_per), :],
                    dst_ref=xrows_ref.at[j],
                    send_sem=send_sems.at[j],
                    recv_sem=recv_sems.at[j],
                    device_id=(j,),
                    device_id_type=pl.DeviceIdType.MESH,
                )
                recv.wait_recv()

            w_dma(j, j % W_SLOTS).wait()

            if _VARIANT == "full":
                wq = wbuf_ref[j % W_SLOTS].astype(jnp.float8_e5m2)
                out_ref[:, :] += jnp.dot(
                    xrows_ref[j], wq, preferred_element_type=jnp.float32,
                )
            elif _VARIANT == "cast":
                wq = wbuf_ref[j % W_SLOTS].astype(jnp.float8_e5m2)
                out_ref[0, pl.ds(0, k_shard)] += wq[0, pl.ds(0, k_shard)].astype(
                    jnp.float32
                )

            if j + W_SLOTS < N_DEV:
                w_dma(j + W_SLOTS, j % W_SLOTS).start()

        scale = sx_ref[0] * sw_ref[0]
        out_ref[:, :] = jnp.maximum(out_ref[:, :] * scale, 0.0)

        for j in range(N_DEV):
            @pl.when(j != me)
            def _(j=j):
                send = pltpu.make_async_remote_copy(
                    src_ref=xq_ref.at[pl.ds(j * m_per, m_per), :],
                    dst_ref=xrows_ref.at[me],
                    send_sem=send_sems.at[j],
                    recv_sem=recv_sems.at[me],
                    device_id=(j,),
                    device_id_type=pl.DeviceIdType.MESH,
                )
                send.wait_send()

    return pl.pallas_call(
        body,
        out_shape=jax.ShapeDtypeStruct((m_per, n), jnp.float32),
        in_specs=[
            pl.BlockSpec(memory_space=pltpu.VMEM),
            pl.BlockSpec(memory_space=pl.ANY),
            pl.BlockSpec(memory_space=pltpu.SMEM),
            pl.BlockSpec(memory_space=pltpu.SMEM),
        ],
        out_specs=pl.BlockSpec(memory_space=pltpu.VMEM),
        scratch_shapes=[
            pltpu.VMEM((m, k_shard), jnp.float8_e4m3fn),
            pltpu.VMEM((N_DEV, m_per, k_shard), jnp.float8_e4m3fn),
            pltpu.VMEM((W_SLOTS, m_per, n), jnp.float32),
            pltpu.SemaphoreType.DMA((N_DEV,)),
            pltpu.SemaphoreType.DMA((N_DEV,)),
            pltpu.SemaphoreType.DMA((W_SLOTS,)),
        ],
    )(x, w_mat, scale_x, scale_w)
